# baseline (speedup 1.0000x reference)
import jax
import jax.numpy as jnp
from jax import lax
from jax.experimental import pallas as pl
from jax.experimental.pallas import tpu as pltpu
from jax.experimental.pallas import tpu_sc as plsc

D = 128
LANES = 16
NC, NS = 2, 16
NW = NC * NS
G = 128
R = 4
S = 10
SG = S * G
NV = 7
NT = NV * NV * NV


def _body(xt_hbm, wd_hbm, ww_hbm, wm_hbm, out_hbm, t_hbm,
          wd_v, ww_v, wm_v, t_v, idxd, idxw, idxm, idxc0, idxc1,
          rows0, rows1, rows2, rows3,
          gsem0, gsem1, gsem2, gsem3, wsem0, wsem1, wsem2, wsem3):
    n_tok = out_hbm.shape[0]
    per_w = n_tok // NW
    n_groups = per_w // G
    n_pairs = n_groups // (2 * S)
    wid = lax.axis_index("s") * NC + lax.axis_index("c")
    base0 = wid * per_w
    rows = (rows0, rows1, rows2, rows3)
    gsem = (gsem0, gsem1, gsem2, gsem3)
    wsem = (wsem0, wsem1, wsem2, wsem3)
    idxc = (idxc0, idxc1)

    pltpu.sync_copy(wd_hbm.at[pl.ds(0, NV)], wd_v)
    pltpu.sync_copy(ww_hbm.at[pl.ds(0, NV)], ww_v)
    pltpu.sync_copy(wm_hbm.at[pl.ds(0, NV)], wm_v)

    def build_dw(dw, _):
        d = dw // NV
        w = dw - d * NV
        for m in range(NV):
            r = dw * NV + m
            for f in range(D // LANES):
                s = pl.ds(f * LANES, LANES)
                t_v[r, s] = wd_v[d, s] + ww_v[w, s] + wm_v[m, s]
        return ()

    lax.fori_loop(0, NV * NV, build_dw, (), unroll=False)
    pltpu.sync_copy(t_v, t_hbm)

    def wait_gather(s):
        pltpu.make_async_copy(
            t_hbm.at[idxc0.at[pl.ds(0, G)]], rows[s], gsem[s]).wait()

    def wait_write(s):
        pltpu.make_async_copy(
            rows[s], out_hbm.at[pl.ds(base0, G)], wsem[s]).wait()

    def pair_body(k2, _):
        for q in range(2):
            ksup = k2 * 2 + q
            tbase = base0 + ksup * SG
            pltpu.sync_copy(xt_hbm.at[pl.ds(tbase, SG)], idxd)
            pltpu.sync_copy(xt_hbm.at[pl.ds(n_tok + tbase, SG)], idxw)
            pltpu.sync_copy(xt_hbm.at[pl.ds(2 * n_tok + tbase, SG)], idxm)
            for i in range(SG // LANES):
                s = pl.ds(i * LANES, LANES)
                idxc[q][s] = (idxd[s] * NV + idxw[s]) * NV + idxm[s]
            for i in range(S):
                gi = q * S + i
                g = k2 * 2 * S + gi
                slot = gi % R
                @pl.when(g >= R)
                def _():
                    wait_write(slot)

                pltpu.async_copy(
                    t_hbm.at[idxc[q].at[pl.ds(i * G, G)]],
                    rows[slot], gsem[slot])

                pslot = (gi - 1) % R
                @pl.when(g >= 1)
                def _():
                    wait_gather(pslot)
                    pltpu.async_copy(
                        rows[pslot],
                        out_hbm.at[pl.ds(base0 + (g - 1) * G, G)],
                        wsem[pslot])
        return ()

    lax.fori_loop(0, n_pairs, pair_body, (), unroll=False)
    last_slot = (n_groups - 1) % R
    wait_gather(last_slot)
    pltpu.async_copy(
        rows[last_slot],
        out_hbm.at[pl.ds(base0 + (n_groups - 1) * G, G)], wsem[last_slot])
    for s in range(R):
        wait_write(s)


def kernel(x, w_day, w_week, w_month):
    b, l, _ = x.shape
    n_tok = b * l
    xt = x.transpose(2, 0, 1).reshape(3 * n_tok)
    mesh = plsc.VectorSubcoreMesh(core_axis_name="c", subcore_axis_name="s",
                                  num_cores=NC, num_subcores=NS)
    run = pl.kernel(
        _body,
        out_type=(jax.ShapeDtypeStruct((n_tok, D), jnp.float32),
                  jax.ShapeDtypeStruct((NT, D), jnp.float32)),
        mesh=mesh,
        scratch_types=[
            pltpu.VMEM((NV, D), jnp.float32),
            pltpu.VMEM((NV, D), jnp.float32),
            pltpu.VMEM((NV, D), jnp.float32),
            pltpu.VMEM((NT, D), jnp.float32),
            pltpu.VMEM((SG,), jnp.int32),
            pltpu.VMEM((SG,), jnp.int32),
            pltpu.VMEM((SG,), jnp.int32),
            pltpu.VMEM((SG,), jnp.int32),
            pltpu.VMEM((SG,), jnp.int32),
            pltpu.VMEM((G, D), jnp.float32),
            pltpu.VMEM((G, D), jnp.float32),
            pltpu.VMEM((G, D), jnp.float32),
            pltpu.VMEM((G, D), jnp.float32),
            pltpu.SemaphoreType.DMA,
            pltpu.SemaphoreType.DMA,
            pltpu.SemaphoreType.DMA,
            pltpu.SemaphoreType.DMA,
            pltpu.SemaphoreType.DMA,
            pltpu.SemaphoreType.DMA,
            pltpu.SemaphoreType.DMA,
            pltpu.SemaphoreType.DMA,
        ],
    )
    out, _ = run(xt, w_day, w_week, w_month)
    return out.reshape(b, l, D)


# speedup vs baseline: 2.9882x; 2.9882x over previous
import jax
import jax.numpy as jnp
from jax import lax
from jax.experimental import pallas as pl
from jax.experimental.pallas import tpu as pltpu
from jax.experimental.pallas import tpu_sc as plsc

D = 128
LANES = 16
NC, NS = 2, 16
NW = NC * NS
G = 128
R = 4
S = 10
SG = S * G
NV = 7
NT = NV * NV * NV


def _body(xt_hbm, wd_hbm, ww_hbm, wm_hbm, out_hbm,
          t_sh, wd_v, ww_v, wm_v, t_v, idxd, idxw, idxm, idxc0, idxc1,
          rows0, rows1, rows2, rows3,
          gsem0, gsem1, gsem2, gsem3, wsem0, wsem1, wsem2, wsem3):
    n_tok = out_hbm.shape[0]
    per_w = n_tok // NW
    n_groups = per_w // G
    n_pairs = n_groups // (2 * S)
    wid = lax.axis_index("s") * NC + lax.axis_index("c")
    base0 = wid * per_w
    rows = (rows0, rows1, rows2, rows3)
    gsem = (gsem0, gsem1, gsem2, gsem3)
    wsem = (wsem0, wsem1, wsem2, wsem3)
    idxc = (idxc0, idxc1)

    pltpu.sync_copy(wd_hbm.at[pl.ds(0, NV)], wd_v)
    pltpu.sync_copy(ww_hbm.at[pl.ds(0, NV)], ww_v)
    pltpu.sync_copy(wm_hbm.at[pl.ds(0, NV)], wm_v)

    def build_dw(dw, _):
        d = dw // NV
        w = dw - d * NV
        for m in range(NV):
            r = dw * NV + m
            for f in range(D // LANES):
                s = pl.ds(f * LANES, LANES)
                t_v[r, s] = wd_v[d, s] + ww_v[w, s] + wm_v[m, s]
        return ()

    lax.fori_loop(0, NV * NV, build_dw, (), unroll=False)
    pltpu.sync_copy(t_v, t_sh)

    def wait_gather(s):
        pltpu.make_async_copy(
            t_sh.at[idxc0.at[pl.ds(0, G)]], rows[s], gsem[s]).wait()

    def wait_write(s):
        pltpu.make_async_copy(
            rows[s], out_hbm.at[pl.ds(base0, G)], wsem[s]).wait()

    def pair_body(k2, _):
        for q in range(2):
            ksup = k2 * 2 + q
            tbase = base0 + ksup * SG
            pltpu.sync_copy(xt_hbm.at[pl.ds(tbase, SG)], idxd)
            pltpu.sync_copy(xt_hbm.at[pl.ds(n_tok + tbase, SG)], idxw)
            pltpu.sync_copy(xt_hbm.at[pl.ds(2 * n_tok + tbase, SG)], idxm)
            for i in range(SG // LANES):
                s = pl.ds(i * LANES, LANES)
                idxc[q][s] = (idxd[s] * NV + idxw[s]) * NV + idxm[s]
            for i in range(S):
                gi = q * S + i
                g = k2 * 2 * S + gi
                slot = gi % R
                @pl.when(g >= R)
                def _():
                    wait_write(slot)

                pltpu.async_copy(
                    t_sh.at[idxc[q].at[pl.ds(i * G, G)]],
                    rows[slot], gsem[slot])

                pslot = (gi - 1) % R
                @pl.when(g >= 1)
                def _():
                    wait_gather(pslot)
                    pltpu.async_copy(
                        rows[pslot],
                        out_hbm.at[pl.ds(base0 + (g - 1) * G, G)],
                        wsem[pslot])
        return ()

    lax.fori_loop(0, n_pairs, pair_body, (), unroll=False)
    last_slot = (n_groups - 1) % R
    wait_gather(last_slot)
    pltpu.async_copy(
        rows[last_slot],
        out_hbm.at[pl.ds(base0 + (n_groups - 1) * G, G)], wsem[last_slot])
    for s in range(R):
        wait_write(s)


def kernel(x, w_day, w_week, w_month):
    b, l, _ = x.shape
    n_tok = b * l
    xt = x.transpose(2, 0, 1).reshape(3 * n_tok)
    mesh = plsc.VectorSubcoreMesh(core_axis_name="c", subcore_axis_name="s",
                                  num_cores=NC, num_subcores=NS)
    run = pl.kernel(
        _body,
        out_type=jax.ShapeDtypeStruct((n_tok, D), jnp.float32),
        mesh=mesh,
        scratch_types=[
            pltpu.VMEM_SHARED((NT, D), jnp.float32),
            pltpu.VMEM((NV, D), jnp.float32),
            pltpu.VMEM((NV, D), jnp.float32),
            pltpu.VMEM((NV, D), jnp.float32),
            pltpu.VMEM((NT, D), jnp.float32),
            pltpu.VMEM((SG,), jnp.int32),
            pltpu.VMEM((SG,), jnp.int32),
            pltpu.VMEM((SG,), jnp.int32),
            pltpu.VMEM((SG,), jnp.int32),
            pltpu.VMEM((SG,), jnp.int32),
            pltpu.VMEM((G, D), jnp.float32),
            pltpu.VMEM((G, D), jnp.float32),
            pltpu.VMEM((G, D), jnp.float32),
            pltpu.VMEM((G, D), jnp.float32),
            pltpu.SemaphoreType.DMA,
            pltpu.SemaphoreType.DMA,
            pltpu.SemaphoreType.DMA,
            pltpu.SemaphoreType.DMA,
            pltpu.SemaphoreType.DMA,
            pltpu.SemaphoreType.DMA,
            pltpu.SemaphoreType.DMA,
            pltpu.SemaphoreType.DMA,
        ],
    )
    out = run(xt, w_day, w_week, w_month)
    return out.reshape(b, l, D)


# speedup vs baseline: 3.4305x; 1.1480x over previous
import jax
import jax.numpy as jnp
from jax import lax
from jax.experimental import pallas as pl
from jax.experimental.pallas import tpu as pltpu
from jax.experimental.pallas import tpu_sc as plsc

D = 128
LANES = 16
NC, NS = 2, 16
NW = NC * NS
G = 128
R = 4
S = 10
SG = S * G
NV = 7
NT = NV * NV * NV


def _body(xt_hbm, wd_hbm, ww_hbm, wm_hbm, out_hbm,
          t_sh, wd_v, ww_v, wm_v, t_v,
          idxd0, idxd1, idxw0, idxw1, idxm0, idxm1, idxc0, idxc1,
          rows0, rows1, rows2, rows3,
          gsem0, gsem1, gsem2, gsem3, wsem0, wsem1, wsem2, wsem3, isem):
    n_tok = out_hbm.shape[0]
    per_w = n_tok // NW
    n_groups = per_w // G
    n_super = per_w // SG
    n_pairs = n_groups // (2 * S)
    wid = lax.axis_index("s") * NC + lax.axis_index("c")
    base0 = wid * per_w
    rows = (rows0, rows1, rows2, rows3)
    gsem = (gsem0, gsem1, gsem2, gsem3)
    wsem = (wsem0, wsem1, wsem2, wsem3)
    idxd = (idxd0, idxd1)
    idxw = (idxw0, idxw1)
    idxm = (idxm0, idxm1)
    idxc = (idxc0, idxc1)

    def fire_stage(tbase, p):
        pltpu.async_copy(xt_hbm.at[pl.ds(tbase, SG)], idxd[p], isem)
        pltpu.async_copy(xt_hbm.at[pl.ds(n_tok + tbase, SG)], idxw[p], isem)
        pltpu.async_copy(xt_hbm.at[pl.ds(2 * n_tok + tbase, SG)], idxm[p], isem)

    def wait_stage(tbase, p):
        pltpu.make_async_copy(
            xt_hbm.at[pl.ds(tbase, SG)], idxd[p], isem).wait()
        pltpu.make_async_copy(
            xt_hbm.at[pl.ds(n_tok + tbase, SG)], idxw[p], isem).wait()
        pltpu.make_async_copy(
            xt_hbm.at[pl.ds(2 * n_tok + tbase, SG)], idxm[p], isem).wait()

    pltpu.sync_copy(wd_hbm.at[pl.ds(0, NV)], wd_v)
    pltpu.sync_copy(ww_hbm.at[pl.ds(0, NV)], ww_v)
    pltpu.sync_copy(wm_hbm.at[pl.ds(0, NV)], wm_v)

    def build_dw(dw, _):
        d = dw // NV
        w = dw - d * NV
        for m in range(NV):
            r = dw * NV + m
            for f in range(D // LANES):
                s = pl.ds(f * LANES, LANES)
                t_v[r, s] = wd_v[d, s] + ww_v[w, s] + wm_v[m, s]
        return ()

    lax.fori_loop(0, NV * NV, build_dw, (), unroll=False)
    pltpu.sync_copy(t_v, t_sh)

    def wait_gather(s):
        pltpu.make_async_copy(
            t_sh.at[idxc0.at[pl.ds(0, G)]], rows[s], gsem[s]).wait()

    def wait_write(s):
        pltpu.make_async_copy(
            rows[s], out_hbm.at[pl.ds(base0, G)], wsem[s]).wait()

    def pair_body(k2, _):
        for q in range(2):
            ksup = k2 * 2 + q
            tbase = base0 + ksup * SG
            wait_stage(tbase, q)

            @pl.when(ksup + 1 < n_super)
            def _():
                fire_stage(tbase + SG, 1 - q)

            for i in range(SG // LANES):
                s = pl.ds(i * LANES, LANES)
                idxc[q][s] = (idxd[q][s] * NV + idxw[q][s]) * NV + idxm[q][s]
            for i in range(S):
                gi = q * S + i
                g = k2 * 2 * S + gi
                slot = gi % R
                @pl.when(g >= R)
                def _():
                    wait_write(slot)

                pltpu.async_copy(
                    t_sh.at[idxc[q].at[pl.ds(i * G, G)]],
                    rows[slot], gsem[slot])

                pslot = (gi - 1) % R
                @pl.when(g >= 1)
                def _():
                    wait_gather(pslot)
                    pltpu.async_copy(
                        rows[pslot],
                        out_hbm.at[pl.ds(base0 + (g - 1) * G, G)],
                        wsem[pslot])
        return ()

    fire_stage(base0, 0)
    lax.fori_loop(0, n_pairs, pair_body, (), unroll=False)
    last_slot = (n_groups - 1) % R
    wait_gather(last_slot)
    pltpu.async_copy(
        rows[last_slot],
        out_hbm.at[pl.ds(base0 + (n_groups - 1) * G, G)], wsem[last_slot])
    for s in range(R):
        wait_write(s)


def kernel(x, w_day, w_week, w_month):
    b, l, _ = x.shape
    n_tok = b * l
    xt = x.transpose(2, 0, 1).reshape(3 * n_tok)
    mesh = plsc.VectorSubcoreMesh(core_axis_name="c", subcore_axis_name="s",
                                  num_cores=NC, num_subcores=NS)
    run = pl.kernel(
        _body,
        out_type=jax.ShapeDtypeStruct((n_tok, D), jnp.float32),
        mesh=mesh,
        scratch_types=[
            pltpu.VMEM_SHARED((NT, D), jnp.float32),
            pltpu.VMEM((NV, D), jnp.float32),
            pltpu.VMEM((NV, D), jnp.float32),
            pltpu.VMEM((NV, D), jnp.float32),
            pltpu.VMEM((NT, D), jnp.float32),
            pltpu.VMEM((SG,), jnp.int32),
            pltpu.VMEM((SG,), jnp.int32),
            pltpu.VMEM((SG,), jnp.int32),
            pltpu.VMEM((SG,), jnp.int32),
            pltpu.VMEM((SG,), jnp.int32),
            pltpu.VMEM((SG,), jnp.int32),
            pltpu.VMEM((SG,), jnp.int32),
            pltpu.VMEM((SG,), jnp.int32),
            pltpu.VMEM((G, D), jnp.float32),
            pltpu.VMEM((G, D), jnp.float32),
            pltpu.VMEM((G, D), jnp.float32),
            pltpu.VMEM((G, D), jnp.float32),
            pltpu.SemaphoreType.DMA,
            pltpu.SemaphoreType.DMA,
            pltpu.SemaphoreType.DMA,
            pltpu.SemaphoreType.DMA,
            pltpu.SemaphoreType.DMA,
            pltpu.SemaphoreType.DMA,
            pltpu.SemaphoreType.DMA,
            pltpu.SemaphoreType.DMA,
            pltpu.SemaphoreType.DMA,
        ],
    )
    out = run(xt, w_day, w_week, w_month)
    return out.reshape(b, l, D)


# speedup vs baseline: 3.6910x; 1.0759x over previous
import jax
import jax.numpy as jnp
from jax import lax
from jax.experimental import pallas as pl
from jax.experimental.pallas import tpu as pltpu
from jax.experimental.pallas import tpu_sc as plsc

D = 128
LANES = 16
NC, NS = 2, 16
NW = NC * NS
G = 128
R = 5
S = 10
SG = S * G
NV = 7
NT = NV * NV * NV
TB = 24
NTP = 384


def _body(xt_hbm, wd_hbm, ww_hbm, wm_hbm, out_hbm,
          t_sh, wd_v, ww_v, wm_v, t_v,
          idxd0, idxd1, idxw0, idxw1, idxm0, idxm1, idxc0, idxc1,
          rows0, rows1, rows2, rows3, rows4,
          gsem0, gsem1, gsem2, gsem3, gsem4,
          wsem0, wsem1, wsem2, wsem3, wsem4, isem):
    n_tok = out_hbm.shape[0]
    per_w = n_tok // NW
    n_groups = per_w // G
    n_super = per_w // SG
    n_pairs = n_groups // (2 * S)
    wid = lax.axis_index("s") * NC + lax.axis_index("c")
    base0 = wid * per_w
    rows = (rows0, rows1, rows2, rows3, rows4)
    gsem = (gsem0, gsem1, gsem2, gsem3, gsem4)
    wsem = (wsem0, wsem1, wsem2, wsem3, wsem4)
    idxd = (idxd0, idxd1)
    idxw = (idxw0, idxw1)
    idxm = (idxm0, idxm1)
    idxc = (idxc0, idxc1)

    def fire_stage(tbase, p):
        pltpu.async_copy(xt_hbm.at[pl.ds(tbase, SG)], idxd[p], isem)
        pltpu.async_copy(xt_hbm.at[pl.ds(n_tok + tbase, SG)], idxw[p], isem)
        pltpu.async_copy(xt_hbm.at[pl.ds(2 * n_tok + tbase, SG)], idxm[p], isem)

    def wait_stage(tbase, p):
        pltpu.make_async_copy(
            xt_hbm.at[pl.ds(tbase, SG)], idxd[p], isem).wait()
        pltpu.make_async_copy(
            xt_hbm.at[pl.ds(n_tok + tbase, SG)], idxw[p], isem).wait()
        pltpu.make_async_copy(
            xt_hbm.at[pl.ds(2 * n_tok + tbase, SG)], idxm[p], isem).wait()

    sid = lax.axis_index("s")
    pltpu.sync_copy(wd_hbm.at[pl.ds(0, NV)], wd_v)
    pltpu.sync_copy(ww_hbm.at[pl.ds(0, NV)], ww_v)
    pltpu.sync_copy(wm_hbm.at[pl.ds(0, NV)], wm_v)
    r0 = sid * TB

    @pl.when(r0 < NT)
    def _():
        def build_row(j, _):
            r = r0 + j

            @pl.when(r < NT)
            def _():
                d = r // (NV * NV)
                rem = r - d * (NV * NV)
                w = rem // NV
                m = rem - w * NV
                for f in range(D // LANES):
                    s = pl.ds(f * LANES, LANES)
                    t_v[j, s] = wd_v[d, s] + ww_v[w, s] + wm_v[m, s]
            return ()

        lax.fori_loop(0, TB, build_row, (), unroll=False)
        pltpu.sync_copy(t_v, t_sh.at[pl.ds(r0, TB)])

    plsc.subcore_barrier()

    def wait_gather(s):
        pltpu.make_async_copy(
            t_sh.at[idxc0.at[pl.ds(0, G)]], rows[s], gsem[s]).wait()

    def wait_write(s):
        pltpu.make_async_copy(
            rows[s], out_hbm.at[pl.ds(base0, G)], wsem[s]).wait()

    def pair_body(k2, _):
        for q in range(2):
            ksup = k2 * 2 + q
            tbase = base0 + ksup * SG
            wait_stage(tbase, q)

            @pl.when(ksup + 1 < n_super)
            def _():
                fire_stage(tbase + SG, 1 - q)

            for i in range(SG // LANES):
                s = pl.ds(i * LANES, LANES)
                idxc[q][s] = (idxd[q][s] * NV + idxw[q][s]) * NV + idxm[q][s]
            for i in range(S):
                gi = q * S + i
                g = k2 * 2 * S + gi
                slot = gi % R
                @pl.when(g >= R)
                def _():
                    wait_write(slot)

                pltpu.async_copy(
                    t_sh.at[idxc[q].at[pl.ds(i * G, G)]],
                    rows[slot], gsem[slot])

                pslot = (gi - 1) % R
                @pl.when(g >= 1)
                def _():
                    wait_gather(pslot)
                    pltpu.async_copy(
                        rows[pslot],
                        out_hbm.at[pl.ds(base0 + (g - 1) * G, G)],
                        wsem[pslot])
        return ()

    fire_stage(base0, 0)
    lax.fori_loop(0, n_pairs, pair_body, (), unroll=False)
    last_slot = (n_groups - 1) % R
    wait_gather(last_slot)
    pltpu.async_copy(
        rows[last_slot],
        out_hbm.at[pl.ds(base0 + (n_groups - 1) * G, G)], wsem[last_slot])
    for s in range(R):
        wait_write(s)


def kernel(x, w_day, w_week, w_month):
    b, l, _ = x.shape
    n_tok = b * l
    xt = x.transpose(2, 0, 1).reshape(3 * n_tok)
    mesh = plsc.VectorSubcoreMesh(core_axis_name="c", subcore_axis_name="s",
                                  num_cores=NC, num_subcores=NS)
    run = pl.kernel(
        _body,
        out_type=jax.ShapeDtypeStruct((n_tok, D), jnp.float32),
        mesh=mesh,
        scratch_types=[
            pltpu.VMEM_SHARED((NTP, D), jnp.float32),
            pltpu.VMEM((NV, D), jnp.float32),
            pltpu.VMEM((NV, D), jnp.float32),
            pltpu.VMEM((NV, D), jnp.float32),
            pltpu.VMEM((TB, D), jnp.float32),
            pltpu.VMEM((SG,), jnp.int32),
            pltpu.VMEM((SG,), jnp.int32),
            pltpu.VMEM((SG,), jnp.int32),
            pltpu.VMEM((SG,), jnp.int32),
            pltpu.VMEM((SG,), jnp.int32),
            pltpu.VMEM((SG,), jnp.int32),
            pltpu.VMEM((SG,), jnp.int32),
            pltpu.VMEM((SG,), jnp.int32),
            pltpu.VMEM((G, D), jnp.float32),
            pltpu.VMEM((G, D), jnp.float32),
            pltpu.VMEM((G, D), jnp.float32),
            pltpu.VMEM((G, D), jnp.float32),
            pltpu.VMEM((G, D), jnp.float32),
            pltpu.SemaphoreType.DMA,
            pltpu.SemaphoreType.DMA,
            pltpu.SemaphoreType.DMA,
            pltpu.SemaphoreType.DMA,
            pltpu.SemaphoreType.DMA,
            pltpu.SemaphoreType.DMA,
            pltpu.SemaphoreType.DMA,
            pltpu.SemaphoreType.DMA,
            pltpu.SemaphoreType.DMA,
            pltpu.SemaphoreType.DMA,
            pltpu.SemaphoreType.DMA,
        ],
    )
    out = run(xt, w_day, w_week, w_month)
    return out.reshape(b, l, D)
